# baseline (speedup 1.0000x reference)
import functools

import jax
import jax.numpy as jnp
from jax import lax
from jax.experimental import pallas as pl
from jax.experimental.pallas import tpu as pltpu
from jax.experimental.pallas import tpu_sc as plsc

N = 10000
E = 320000
D = 128
NUM_BLOCKS = 3
G = 64

NC, NS = 2, 16
NW = NC * NS
E_PER_W = E // NW
N_PAD = 10240
ROWS_PER_S = N_PAD // NS
WB = 64
K = 64
EPW_PAD = 10240
CH = EPW_PAD // K
PADE = EPW_PAD - E_PER_W

RB = 1000
GRID = N // RB
DGB = 32

_mesh = plsc.VectorSubcoreMesh(
    core_axis_name="c", subcore_axis_name="s", num_cores=NC, num_subcores=NS)


@functools.partial(
    pl.kernel,
    out_type=jax.ShapeDtypeStruct((NW * N,), jnp.float32),
    mesh=_mesh,
    compiler_params=pltpu.CompilerParams(needs_layout_passes=False),
    scratch_types=[
        pltpu.VMEM((E_PER_W,), jnp.int32),
        pltpu.VMEM((N,), jnp.float32),
    ],
)
def _deg_sc(dst_hbm, zeros_hbm, out_hbm, dbuf, acc):
    c = lax.axis_index("c")
    s = lax.axis_index("s")
    wid = c * NS + s
    pltpu.sync_copy(zeros_hbm, acc)
    pltpu.sync_copy(dst_hbm.at[pl.ds(wid * E_PER_W, E_PER_W)], dbuf)
    ones16 = jnp.ones((16,), jnp.float32)

    def body(j, carry):
        idx = dbuf[pl.ds(j * 16, 16)]
        plsc.addupdate_scatter(acc, [idx], ones16)
        return carry

    lax.fori_loop(0, E_PER_W // 16, body, 0)
    pltpu.sync_copy(acc, out_hbm.at[pl.ds(wid * N, N)])


@functools.partial(
    pl.kernel,
    out_type=jax.ShapeDtypeStruct((NC, N_PAD, D), jnp.float32),
    mesh=_mesh,
    compiler_params=pltpu.CompilerParams(needs_layout_passes=False),
    scratch_types=[
        pltpu.VMEM((CH, K), jnp.int32),
        pltpu.VMEM((K,), jnp.int32),
        pltpu.VMEM((K,), jnp.int32),
        pltpu.VMEM((K, D), jnp.float32),
        pltpu.VMEM((K, D), jnp.float32),
        pltpu.VMEM_SHARED((N_PAD, D), jnp.float32),
        pltpu.SemaphoreType.DMA,
        pltpu.SemaphoreType.DMA,
    ],
)
def _agg_sc(h_hbm, src_hbm, dst_hbm, zrows_hbm, out_hbm,
            sbuf, d0, d1, rows0, rows1, accum, sem0, sem1):
    c = lax.axis_index("c")
    s = lax.axis_index("s")
    wid = c * NS + s
    stage = rows0.at[pl.ds(0, WB)]

    pltpu.sync_copy(zrows_hbm, stage)

    def zb(t, carry):
        pltpu.sync_copy(stage, accum.at[pl.ds(s * ROWS_PER_S + t * WB, WB)])
        return carry

    lax.fori_loop(0, ROWS_PER_S // WB, zb, 0)

    pltpu.sync_copy(src_hbm.at[wid], sbuf)
    plsc.subcore_barrier()

    pltpu.async_copy(h_hbm.at[sbuf.at[0]], rows0, sem0)
    pltpu.sync_copy(dst_hbm.at[wid, 0], d0)

    def body(t2, carry):
        t = t2 * 2
        g1 = pltpu.async_copy(h_hbm.at[sbuf.at[t + 1]], rows1, sem1)
        pltpu.sync_copy(dst_hbm.at[wid, t + 1], d1)
        pltpu.make_async_copy(h_hbm.at[sbuf.at[t]], rows0, sem0).wait()
        pltpu.sync_copy(rows0, accum.at[d0], add=True)

        @pl.when(t2 < CH // 2 - 1)
        def _():
            pltpu.async_copy(h_hbm.at[sbuf.at[t + 2]], rows0, sem0)
            pltpu.sync_copy(dst_hbm.at[wid, t + 2], d0)

        g1.wait()
        pltpu.sync_copy(rows1, accum.at[d1], add=True)
        return carry

    lax.fori_loop(0, CH // 2, body, 0)
    plsc.subcore_barrier()

    def wb(t, carry):
        r0 = s * ROWS_PER_S + t * WB
        pltpu.sync_copy(accum.at[pl.ds(r0, WB)], stage)
        pltpu.sync_copy(stage, out_hbm.at[c, pl.ds(r0, WB)])
        return carry

    lax.fori_loop(0, ROWS_PER_S // WB, wb, 0)


def _proj_body(x_ref, w_ref, b_ref, o_ref):
    o_ref[...] = (
        jnp.dot(x_ref[...], w_ref[...], preferred_element_type=jnp.float32)
        + b_ref[...])


def _proj(x, w, b):
    return pl.pallas_call(
        _proj_body,
        grid=(GRID,),
        in_specs=[
            pl.BlockSpec((RB, D), lambda i: (i, 0)),
            pl.BlockSpec((D, D), lambda i: (0, 0)),
            pl.BlockSpec((1, D), lambda i: (0, 0)),
        ],
        out_specs=pl.BlockSpec((RB, D), lambda i: (i, 0)),
        out_shape=jax.ShapeDtypeStruct((N, D), jnp.float32),
    )(x, w, b)


def _ln(v, g, b):
    mu = jnp.mean(v, axis=-1, keepdims=True)
    var = jnp.mean((v - mu) ** 2, axis=-1, keepdims=True)
    return (v - mu) * lax.rsqrt(var + 1e-5) * g + b


def _block_body(h_ref, p_ref, degp_ref, bt_ref, wl_ref, wr_ref, bb_ref,
                g_ref, beta_ref, gprev_ref, hout_ref, gout_ref, acc_ref):
    i = pl.program_id(0)
    hb = h_ref[...]
    psum = p_ref[0] + p_ref[1]
    ones_w = jnp.ones((DGB, 1), jnp.float32)
    deg_col = lax.dot_general(
        degp_ref[0], ones_w, (((0,), (0,)), ((), ())),
        preferred_element_type=jnp.float32)
    deg_col = jnp.maximum(deg_col, 1.0)
    agg = psum / deg_col
    node_conv = (
        jnp.dot(hb, wl_ref[...], preferred_element_type=jnp.float32)
        + jnp.dot(agg, wr_ref[...], preferred_element_type=jnp.float32)
        + bb_ref[...])
    onehot = (bt_ref[...] == lax.broadcasted_iota(jnp.int32, (RB, G), 1)
              ).astype(jnp.float32)
    gc = lax.dot_general(
        onehot, node_conv, (((0,), (0,)), ((), ())),
        preferred_element_type=jnp.float32)

    @pl.when(i == 0)
    def _():
        acc_ref[...] = gc

    @pl.when(i > 0)
    def _():
        acc_ref[...] += gc

    gamma = g_ref[...]
    beta = beta_ref[...]
    hout_ref[...] = jnp.maximum(_ln(node_conv + hb, gamma, beta), 0.0)

    @pl.when(i == GRID - 1)
    def _():
        gtot = acc_ref[...] + gprev_ref[...]
        gout_ref[...] = jnp.maximum(_ln(gtot, gamma, beta), 0.0)


def _block(h, p, degp, bt, wl, wr, bb, gamma, beta, gprev):
    return pl.pallas_call(
        _block_body,
        grid=(GRID,),
        in_specs=[
            pl.BlockSpec((RB, D), lambda i: (i, 0)),
            pl.BlockSpec((NC, RB, D), lambda i: (0, i, 0)),
            pl.BlockSpec((1, NW, RB), lambda i: (i, 0, 0)),
            pl.BlockSpec((RB, 1), lambda i: (i, 0)),
            pl.BlockSpec((D, D), lambda i: (0, 0)),
            pl.BlockSpec((D, D), lambda i: (0, 0)),
            pl.BlockSpec((1, D), lambda i: (0, 0)),
            pl.BlockSpec((1, D), lambda i: (0, 0)),
            pl.BlockSpec((1, D), lambda i: (0, 0)),
            pl.BlockSpec((G, D), lambda i: (0, 0)),
        ],
        out_specs=[
            pl.BlockSpec((RB, D), lambda i: (i, 0)),
            pl.BlockSpec((G, D), lambda i: (0, 0)),
        ],
        out_shape=[
            jax.ShapeDtypeStruct((N, D), jnp.float32),
            jax.ShapeDtypeStruct((G, D), jnp.float32),
        ],
        scratch_shapes=[pltpu.VMEM((G, D), jnp.float32)],
    )(h, p, degp, bt, wl, wr, bb, gamma, beta, gprev)


def kernel(x, edge_index, batch, W_fc, b_fc, Wl, Wr, bb, gamma, beta):
    src = edge_index[0].astype(jnp.int32)
    dst = edge_index[1].astype(jnp.int32)
    bt = batch.astype(jnp.int32).reshape(N, 1)
    zrows = jnp.zeros((WB, D), jnp.float32)
    zdeg = jnp.zeros((N,), jnp.float32)

    degp = _deg_sc(dst, zdeg).reshape(NW, GRID, RB).transpose(1, 0, 2)
    src3 = jnp.pad(src.reshape(NW, E_PER_W), ((0, 0), (0, PADE)),
                   constant_values=0).reshape(NW, CH, K)
    pad_rows = jnp.broadcast_to(
        (N + jnp.arange(NW, dtype=jnp.int32))[:, None], (NW, PADE))
    dst3 = jnp.concatenate(
        [dst.reshape(NW, E_PER_W), pad_rows], axis=1).reshape(NW, CH, K)
    h = _proj(x, W_fc, b_fc.reshape(1, D))
    g = jnp.zeros((G, D), jnp.float32)
    for i in range(NUM_BLOCKS):
        p = _agg_sc(h, src3, dst3, zrows)
        h, g = _block(h, p, degp, bt, Wl[i], Wr[i], bb[i].reshape(1, D),
                      gamma[i].reshape(1, D), beta[i].reshape(1, D), g)
    return h, g


# speedup vs baseline: 2.2316x; 2.2316x over previous
import functools

import jax
import jax.numpy as jnp
from jax import lax
from jax.experimental import pallas as pl
from jax.experimental.pallas import tpu as pltpu
from jax.experimental.pallas import tpu_sc as plsc

N = 10000
E = 320000
D = 128
NUM_BLOCKS = 3
G = 64

NC, NS = 2, 16
NW = NC * NS
E_PER_W = E // NW
N_PAD = 10240
ROWS_PER_S = N_PAD // NS
WB = 64
K = 50
CH = E_PER_W // K

RB = 1000
GRID = N // RB
DGB = 32

_mesh = plsc.VectorSubcoreMesh(
    core_axis_name="c", subcore_axis_name="s", num_cores=NC, num_subcores=NS)


@functools.partial(
    pl.kernel,
    out_type=jax.ShapeDtypeStruct((NW * N,), jnp.float32),
    mesh=_mesh,
    compiler_params=pltpu.CompilerParams(needs_layout_passes=False),
    scratch_types=[
        pltpu.VMEM((E_PER_W,), jnp.int32),
        pltpu.VMEM((N,), jnp.float32),
    ],
)
def _deg_sc(dst_hbm, zeros_hbm, out_hbm, dbuf, acc):
    c = lax.axis_index("c")
    s = lax.axis_index("s")
    wid = c * NS + s
    pltpu.sync_copy(zeros_hbm, acc)
    pltpu.sync_copy(dst_hbm.at[pl.ds(wid * E_PER_W, E_PER_W)], dbuf)
    ones16 = jnp.ones((16,), jnp.float32)

    def body(j, carry):
        idx = dbuf[pl.ds(j * 16, 16)]
        plsc.addupdate_scatter(acc, [idx], ones16)
        return carry

    lax.fori_loop(0, E_PER_W // 16, body, 0)
    pltpu.sync_copy(acc, out_hbm.at[pl.ds(wid * N, N)])


@functools.partial(
    pl.kernel,
    out_type=jax.ShapeDtypeStruct((NC, N_PAD, D), jnp.float32),
    mesh=_mesh,
    compiler_params=pltpu.CompilerParams(needs_layout_passes=False),
    scratch_types=[
        pltpu.VMEM((CH, K), jnp.int32),
        pltpu.VMEM((K,), jnp.int32),
        pltpu.VMEM((K,), jnp.int32),
        pltpu.VMEM((K, D), jnp.float32),
        pltpu.VMEM((K, D), jnp.float32),
        pltpu.VMEM_SHARED((N_PAD, D), jnp.float32),
        pltpu.SemaphoreType.DMA,
        pltpu.SemaphoreType.DMA,
    ],
)
def _agg_sc(h_hbm, src_hbm, dst_hbm, zrows_hbm, out_hbm,
            sbuf, d0, d1, rows0, rows1, accum, sem0, sem1):
    c = lax.axis_index("c")
    s = lax.axis_index("s")
    wid = c * NS + s
    stage = rows0.at[pl.ds(0, WB)]

    pltpu.sync_copy(zrows_hbm, stage)

    def zb(t, carry):
        pltpu.sync_copy(stage, accum.at[pl.ds(s * ROWS_PER_S + t * WB, WB)])
        return carry

    lax.fori_loop(0, ROWS_PER_S // WB, zb, 0)

    pltpu.sync_copy(src_hbm.at[wid], sbuf)
    plsc.subcore_barrier()

    pltpu.async_copy(h_hbm.at[sbuf.at[0]], rows0, sem0)
    pltpu.sync_copy(dst_hbm.at[wid, 0], d0)

    def body(t2, carry):
        t = t2 * 2
        g1 = pltpu.async_copy(h_hbm.at[sbuf.at[t + 1]], rows1, sem1)
        pltpu.sync_copy(dst_hbm.at[wid, t + 1], d1)
        pltpu.make_async_copy(h_hbm.at[sbuf.at[t]], rows0, sem0).wait()
        pltpu.sync_copy(rows0, accum.at[d0], add=True)

        @pl.when(t2 < CH // 2 - 1)
        def _():
            pltpu.async_copy(h_hbm.at[sbuf.at[t + 2]], rows0, sem0)
            pltpu.sync_copy(dst_hbm.at[wid, t + 2], d0)

        g1.wait()
        pltpu.sync_copy(rows1, accum.at[d1], add=True)
        return carry

    lax.fori_loop(0, CH // 2, body, 0)
    plsc.subcore_barrier()

    def wb(t, carry):
        r0 = s * ROWS_PER_S + t * WB
        pltpu.sync_copy(accum.at[pl.ds(r0, WB)], stage)
        pltpu.sync_copy(stage, out_hbm.at[c, pl.ds(r0, WB)])
        return carry

    lax.fori_loop(0, ROWS_PER_S // WB, wb, 0)


def _proj_body(x_ref, w_ref, b_ref, o_ref):
    o_ref[...] = (
        jnp.dot(x_ref[...], w_ref[...], preferred_element_type=jnp.float32)
        + b_ref[...])


def _proj(x, w, b):
    return pl.pallas_call(
        _proj_body,
        grid=(GRID,),
        in_specs=[
            pl.BlockSpec((RB, D), lambda i: (i, 0)),
            pl.BlockSpec((D, D), lambda i: (0, 0)),
            pl.BlockSpec((1, D), lambda i: (0, 0)),
        ],
        out_specs=pl.BlockSpec((RB, D), lambda i: (i, 0)),
        out_shape=jax.ShapeDtypeStruct((N, D), jnp.float32),
    )(x, w, b)


def _ln(v, g, b):
    mu = jnp.mean(v, axis=-1, keepdims=True)
    var = jnp.mean((v - mu) ** 2, axis=-1, keepdims=True)
    return (v - mu) * lax.rsqrt(var + 1e-5) * g + b


def _block_body(h_ref, p_ref, degp_ref, bt_ref, wl_ref, wr_ref, bb_ref,
                g_ref, beta_ref, gprev_ref, hout_ref, gout_ref, acc_ref):
    i = pl.program_id(0)
    hb = h_ref[...]
    psum = p_ref[0] + p_ref[1]
    ones_w = jnp.ones((DGB, 1), jnp.float32)
    deg_col = lax.dot_general(
        degp_ref[0], ones_w, (((0,), (0,)), ((), ())),
        preferred_element_type=jnp.float32)
    deg_col = jnp.maximum(deg_col, 1.0)
    agg = psum / deg_col
    node_conv = (
        jnp.dot(hb, wl_ref[...], preferred_element_type=jnp.float32)
        + jnp.dot(agg, wr_ref[...], preferred_element_type=jnp.float32)
        + bb_ref[...])
    onehot = (bt_ref[...] == lax.broadcasted_iota(jnp.int32, (RB, G), 1)
              ).astype(jnp.float32)
    gc = lax.dot_general(
        onehot, node_conv, (((0,), (0,)), ((), ())),
        preferred_element_type=jnp.float32)

    @pl.when(i == 0)
    def _():
        acc_ref[...] = gc

    @pl.when(i > 0)
    def _():
        acc_ref[...] += gc

    gamma = g_ref[...]
    beta = beta_ref[...]
    hout_ref[...] = jnp.maximum(_ln(node_conv + hb, gamma, beta), 0.0)

    @pl.when(i == GRID - 1)
    def _():
        gtot = acc_ref[...] + gprev_ref[...]
        gout_ref[...] = jnp.maximum(_ln(gtot, gamma, beta), 0.0)


def _block(h, p, degp, bt, wl, wr, bb, gamma, beta, gprev):
    return pl.pallas_call(
        _block_body,
        grid=(GRID,),
        in_specs=[
            pl.BlockSpec((RB, D), lambda i: (i, 0)),
            pl.BlockSpec((NC, RB, D), lambda i: (0, i, 0)),
            pl.BlockSpec((1, NW, RB), lambda i: (i, 0, 0)),
            pl.BlockSpec((RB, 1), lambda i: (i, 0)),
            pl.BlockSpec((D, D), lambda i: (0, 0)),
            pl.BlockSpec((D, D), lambda i: (0, 0)),
            pl.BlockSpec((1, D), lambda i: (0, 0)),
            pl.BlockSpec((1, D), lambda i: (0, 0)),
            pl.BlockSpec((1, D), lambda i: (0, 0)),
            pl.BlockSpec((G, D), lambda i: (0, 0)),
        ],
        out_specs=[
            pl.BlockSpec((RB, D), lambda i: (i, 0)),
            pl.BlockSpec((G, D), lambda i: (0, 0)),
        ],
        out_shape=[
            jax.ShapeDtypeStruct((N, D), jnp.float32),
            jax.ShapeDtypeStruct((G, D), jnp.float32),
        ],
        scratch_shapes=[pltpu.VMEM((G, D), jnp.float32)],
    )(h, p, degp, bt, wl, wr, bb, gamma, beta, gprev)


def kernel(x, edge_index, batch, W_fc, b_fc, Wl, Wr, bb, gamma, beta):
    src = edge_index[0].astype(jnp.int32)
    dst = edge_index[1].astype(jnp.int32)
    bt = batch.astype(jnp.int32).reshape(N, 1)
    zrows = jnp.zeros((WB, D), jnp.float32)
    zdeg = jnp.zeros((N,), jnp.float32)

    degp = _deg_sc(dst, zdeg).reshape(NW, GRID, RB).transpose(1, 0, 2)
    src3 = src.reshape(NW, CH, K)
    dst3 = dst.reshape(NW, CH, K)
    h = _proj(x, W_fc, b_fc.reshape(1, D))
    g = jnp.zeros((G, D), jnp.float32)
    for i in range(NUM_BLOCKS):
        p = _agg_sc(h, src3, dst3, zrows)
        h, g = _block(h, p, degp, bt, Wl[i], Wr[i], bb[i].reshape(1, D),
                      gamma[i].reshape(1, D), beta[i].reshape(1, D), g)
    return h, g


# speedup vs baseline: 2.6251x; 1.1763x over previous
import functools

import jax
import jax.numpy as jnp
from jax import lax
from jax.experimental import pallas as pl
from jax.experimental.pallas import tpu as pltpu
from jax.experimental.pallas import tpu_sc as plsc

N = 10000
E = 320000
D = 128
NUM_BLOCKS = 3
G = 64

NC, NS = 2, 16
NW = NC * NS
E_PER_W = E // NW
N_PAD = 10240
ROWS_PER_S = N_PAD // NS
WB = 64
K = 50
CH = E_PER_W // K

RB = 1000
GRID = N // RB
DGB = 32

_mesh = plsc.VectorSubcoreMesh(
    core_axis_name="c", subcore_axis_name="s", num_cores=NC, num_subcores=NS)


@functools.partial(
    pl.kernel,
    out_type=jax.ShapeDtypeStruct((NW * N,), jnp.float32),
    mesh=_mesh,
    compiler_params=pltpu.CompilerParams(needs_layout_passes=False),
    scratch_types=[
        pltpu.VMEM((E_PER_W,), jnp.int32),
        pltpu.VMEM((N,), jnp.float32),
    ],
)
def _deg_sc(dst_hbm, zeros_hbm, out_hbm, dbuf, acc):
    c = lax.axis_index("c")
    s = lax.axis_index("s")
    wid = c * NS + s
    pltpu.sync_copy(zeros_hbm, acc)
    pltpu.sync_copy(dst_hbm.at[pl.ds(wid * E_PER_W, E_PER_W)], dbuf)
    ones16 = jnp.ones((16,), jnp.float32)

    def body(j, carry):
        idx = dbuf[pl.ds(j * 16, 16)]
        plsc.addupdate_scatter(acc, [idx], ones16)
        return carry

    lax.fori_loop(0, E_PER_W // 16, body, 0)
    pltpu.sync_copy(acc, out_hbm.at[pl.ds(wid * N, N)])


@functools.partial(
    pl.kernel,
    out_type=jax.ShapeDtypeStruct((NC, N_PAD, D), jnp.float32),
    mesh=_mesh,
    compiler_params=pltpu.CompilerParams(needs_layout_passes=False),
    scratch_types=[
        pltpu.VMEM((CH, K), jnp.int32),
        pltpu.VMEM((K,), jnp.int32),
        pltpu.VMEM((K,), jnp.int32),
        pltpu.VMEM((K, D), jnp.float32),
        pltpu.VMEM((K, D), jnp.float32),
        pltpu.VMEM_SHARED((N_PAD, D), jnp.float32),
        pltpu.SemaphoreType.DMA,
        pltpu.SemaphoreType.DMA,
        pltpu.SemaphoreType.DMA,
        pltpu.SemaphoreType.DMA,
    ],
)
def _agg_sc(h_hbm, src_hbm, dst_hbm, zrows_hbm, out_hbm,
            sbuf, d0, d1, rows0, rows1, accum, sem0, sem1, semd0, semd1):
    c = lax.axis_index("c")
    s = lax.axis_index("s")
    wid = c * NS + s
    stage = rows0.at[pl.ds(0, WB)]

    pltpu.sync_copy(zrows_hbm, stage)

    def zb(t, carry):
        pltpu.sync_copy(stage, accum.at[pl.ds(s * ROWS_PER_S + t * WB, WB)])
        return carry

    lax.fori_loop(0, ROWS_PER_S // WB, zb, 0)

    pltpu.sync_copy(src_hbm.at[wid], sbuf)
    plsc.subcore_barrier()

    pltpu.async_copy(h_hbm.at[sbuf.at[0]], rows0, sem0)
    pltpu.async_copy(dst_hbm.at[wid, 0], d0, semd0)
    pltpu.async_copy(dst_hbm.at[wid, 1], d1, semd1)

    def body(t2, carry):
        t = t2 * 2
        g1 = pltpu.async_copy(h_hbm.at[sbuf.at[t + 1]], rows1, sem1)
        pltpu.make_async_copy(dst_hbm.at[wid, t], d0, semd0).wait()
        pltpu.make_async_copy(h_hbm.at[sbuf.at[t]], rows0, sem0).wait()
        pltpu.sync_copy(rows0, accum.at[d0], add=True)

        @pl.when(t2 < CH // 2 - 1)
        def _():
            pltpu.async_copy(h_hbm.at[sbuf.at[t + 2]], rows0, sem0)
            pltpu.async_copy(dst_hbm.at[wid, t + 2], d0, semd0)

        pltpu.make_async_copy(dst_hbm.at[wid, t + 1], d1, semd1).wait()
        g1.wait()
        pltpu.sync_copy(rows1, accum.at[d1], add=True)

        @pl.when(t2 < CH // 2 - 1)
        def _():
            pltpu.async_copy(dst_hbm.at[wid, t + 3], d1, semd1)

        return carry

    lax.fori_loop(0, CH // 2, body, 0)
    plsc.subcore_barrier()

    def wb(t, carry):
        r0 = s * ROWS_PER_S + t * WB
        pltpu.sync_copy(accum.at[pl.ds(r0, WB)], stage)
        pltpu.sync_copy(stage, out_hbm.at[c, pl.ds(r0, WB)])
        return carry

    lax.fori_loop(0, ROWS_PER_S // WB, wb, 0)


def _proj_body(x_ref, w_ref, b_ref, o_ref):
    o_ref[...] = (
        jnp.dot(x_ref[...], w_ref[...], preferred_element_type=jnp.float32)
        + b_ref[...])


def _proj(x, w, b):
    return pl.pallas_call(
        _proj_body,
        grid=(GRID,),
        in_specs=[
            pl.BlockSpec((RB, D), lambda i: (i, 0)),
            pl.BlockSpec((D, D), lambda i: (0, 0)),
            pl.BlockSpec((1, D), lambda i: (0, 0)),
        ],
        out_specs=pl.BlockSpec((RB, D), lambda i: (i, 0)),
        out_shape=jax.ShapeDtypeStruct((N, D), jnp.float32),
    )(x, w, b)


def _ln(v, g, b):
    mu = jnp.mean(v, axis=-1, keepdims=True)
    var = jnp.mean((v - mu) ** 2, axis=-1, keepdims=True)
    return (v - mu) * lax.rsqrt(var + 1e-5) * g + b


def _block_body(h_ref, p_ref, degp_ref, bt_ref, wl_ref, wr_ref, bb_ref,
                g_ref, beta_ref, gprev_ref, hout_ref, gout_ref, acc_ref):
    i = pl.program_id(0)
    hb = h_ref[...]
    psum = p_ref[0] + p_ref[1]
    ones_w = jnp.ones((DGB, 1), jnp.float32)
    deg_col = lax.dot_general(
        degp_ref[0], ones_w, (((0,), (0,)), ((), ())),
        preferred_element_type=jnp.float32)
    deg_col = jnp.maximum(deg_col, 1.0)
    agg = psum / deg_col
    node_conv = (
        jnp.dot(hb, wl_ref[...], preferred_element_type=jnp.float32)
        + jnp.dot(agg, wr_ref[...], preferred_element_type=jnp.float32)
        + bb_ref[...])
    onehot = (bt_ref[...] == lax.broadcasted_iota(jnp.int32, (RB, G), 1)
              ).astype(jnp.float32)
    gc = lax.dot_general(
        onehot, node_conv, (((0,), (0,)), ((), ())),
        preferred_element_type=jnp.float32)

    @pl.when(i == 0)
    def _():
        acc_ref[...] = gc

    @pl.when(i > 0)
    def _():
        acc_ref[...] += gc

    gamma = g_ref[...]
    beta = beta_ref[...]
    hout_ref[...] = jnp.maximum(_ln(node_conv + hb, gamma, beta), 0.0)

    @pl.when(i == GRID - 1)
    def _():
        gtot = acc_ref[...] + gprev_ref[...]
        gout_ref[...] = jnp.maximum(_ln(gtot, gamma, beta), 0.0)


def _block(h, p, degp, bt, wl, wr, bb, gamma, beta, gprev):
    return pl.pallas_call(
        _block_body,
        grid=(GRID,),
        in_specs=[
            pl.BlockSpec((RB, D), lambda i: (i, 0)),
            pl.BlockSpec((NC, RB, D), lambda i: (0, i, 0)),
            pl.BlockSpec((1, NW, RB), lambda i: (i, 0, 0)),
            pl.BlockSpec((RB, 1), lambda i: (i, 0)),
            pl.BlockSpec((D, D), lambda i: (0, 0)),
            pl.BlockSpec((D, D), lambda i: (0, 0)),
            pl.BlockSpec((1, D), lambda i: (0, 0)),
            pl.BlockSpec((1, D), lambda i: (0, 0)),
            pl.BlockSpec((1, D), lambda i: (0, 0)),
            pl.BlockSpec((G, D), lambda i: (0, 0)),
        ],
        out_specs=[
            pl.BlockSpec((RB, D), lambda i: (i, 0)),
            pl.BlockSpec((G, D), lambda i: (0, 0)),
        ],
        out_shape=[
            jax.ShapeDtypeStruct((N, D), jnp.float32),
            jax.ShapeDtypeStruct((G, D), jnp.float32),
        ],
        scratch_shapes=[pltpu.VMEM((G, D), jnp.float32)],
    )(h, p, degp, bt, wl, wr, bb, gamma, beta, gprev)


def kernel(x, edge_index, batch, W_fc, b_fc, Wl, Wr, bb, gamma, beta):
    src = edge_index[0].astype(jnp.int32)
    dst = edge_index[1].astype(jnp.int32)
    bt = batch.astype(jnp.int32).reshape(N, 1)
    zrows = jnp.zeros((WB, D), jnp.float32)
    zdeg = jnp.zeros((N,), jnp.float32)

    degp = _deg_sc(dst, zdeg).reshape(NW, GRID, RB).transpose(1, 0, 2)
    src3 = src.reshape(NW, CH, K)
    dst3 = dst.reshape(NW, CH, K)
    h = _proj(x, W_fc, b_fc.reshape(1, D))
    g = jnp.zeros((G, D), jnp.float32)
    for i in range(NUM_BLOCKS):
        p = _agg_sc(h, src3, dst3, zrows)
        h, g = _block(h, p, degp, bt, Wl[i], Wr[i], bb[i].reshape(1, D),
                      gamma[i].reshape(1, D), beta[i].reshape(1, D), g)
    return h, g


# speedup vs baseline: 3.2590x; 1.2415x over previous
import functools

import jax
import jax.numpy as jnp
from jax import lax
from jax.experimental import pallas as pl
from jax.experimental.pallas import tpu as pltpu
from jax.experimental.pallas import tpu_sc as plsc

N = 10000
E = 320000
D = 128
NUM_BLOCKS = 3
G = 64

NC, NS = 2, 16
NW = NC * NS
E_PER_W = E // NW
N_PAD = 10240
ROWS_PER_S = N_PAD // NS
WB = 64
K = 50
CH = E_PER_W // K
NB = 3

RB = 1000
GRID = N // RB
DGB = 32

_mesh = plsc.VectorSubcoreMesh(
    core_axis_name="c", subcore_axis_name="s", num_cores=NC, num_subcores=NS)


@functools.partial(
    pl.kernel,
    out_type=jax.ShapeDtypeStruct((NW * N,), jnp.float32),
    mesh=_mesh,
    compiler_params=pltpu.CompilerParams(needs_layout_passes=False),
    scratch_types=[
        pltpu.VMEM((E_PER_W,), jnp.int32),
        pltpu.VMEM((N,), jnp.float32),
    ],
)
def _deg_sc(dst_hbm, zeros_hbm, out_hbm, dbuf, acc):
    c = lax.axis_index("c")
    s = lax.axis_index("s")
    wid = c * NS + s
    pltpu.sync_copy(zeros_hbm, acc)
    pltpu.sync_copy(dst_hbm.at[pl.ds(wid * E_PER_W, E_PER_W)], dbuf)
    ones16 = jnp.ones((16,), jnp.float32)

    def body(j, carry):
        idx = dbuf[pl.ds(j * 16, 16)]
        plsc.addupdate_scatter(acc, [idx], ones16)
        return carry

    lax.fori_loop(0, E_PER_W // 16, body, 0)
    pltpu.sync_copy(acc, out_hbm.at[pl.ds(wid * N, N)])


@functools.partial(
    pl.kernel,
    out_type=jax.ShapeDtypeStruct((NC, N_PAD, D), jnp.float32),
    mesh=_mesh,
    compiler_params=pltpu.CompilerParams(needs_layout_passes=False),
    scratch_types=[
        pltpu.VMEM((CH, K), jnp.int32),
        [pltpu.VMEM((K,), jnp.int32) for _ in range(NB)],
        [pltpu.VMEM((K, D), jnp.float32) for _ in range(NB)],
        pltpu.VMEM_SHARED((N_PAD, D), jnp.float32),
        [pltpu.SemaphoreType.DMA for _ in range(NB)],
        [pltpu.SemaphoreType.DMA for _ in range(NB)],
    ],
)
def _agg_sc(h_hbm, src_hbm, dst_hbm, zrows_hbm, out_hbm,
            sbuf, dring, rring, accum, gsems, dsems):
    c = lax.axis_index("c")
    s = lax.axis_index("s")
    wid = c * NS + s
    stage = rring[0].at[pl.ds(0, WB)]

    pltpu.sync_copy(zrows_hbm, stage)

    def zb(t, carry):
        pltpu.sync_copy(stage, accum.at[pl.ds(s * ROWS_PER_S + t * WB, WB)])
        return carry

    lax.fori_loop(0, ROWS_PER_S // WB, zb, 0)

    pltpu.sync_copy(src_hbm.at[wid], sbuf)
    plsc.subcore_barrier()

    for k in range(NB):
        pltpu.async_copy(h_hbm.at[sbuf.at[k]], rring[k], gsems[k])
        pltpu.async_copy(dst_hbm.at[wid, k], dring[k], dsems[k])

    def body(i, carry):
        t0 = i * NB
        for k in range(NB):
            t = t0 + k
            pltpu.make_async_copy(dst_hbm.at[wid, t], dring[k],
                                  dsems[k]).wait()
            pltpu.make_async_copy(h_hbm.at[sbuf.at[t]], rring[k],
                                  gsems[k]).wait()
            pltpu.sync_copy(rring[k], accum.at[dring[k]], add=True)

            @pl.when(t + NB < CH)
            def _(t=t, k=k):
                pltpu.async_copy(h_hbm.at[sbuf.at[t + NB]], rring[k],
                                 gsems[k])
                pltpu.async_copy(dst_hbm.at[wid, t + NB], dring[k], dsems[k])
        return carry

    lax.fori_loop(0, CH // NB, body, 0)
    for k in range(CH - NB * (CH // NB)):
        t = NB * (CH // NB) + k
        pltpu.make_async_copy(dst_hbm.at[wid, t], dring[k], dsems[k]).wait()
        pltpu.make_async_copy(h_hbm.at[sbuf.at[t]], rring[k], gsems[k]).wait()
        pltpu.sync_copy(rring[k], accum.at[dring[k]], add=True)
    plsc.subcore_barrier()

    def wb(t, carry):
        r0 = s * ROWS_PER_S + t * WB
        pltpu.sync_copy(accum.at[pl.ds(r0, WB)], stage)
        pltpu.sync_copy(stage, out_hbm.at[c, pl.ds(r0, WB)])
        return carry

    lax.fori_loop(0, ROWS_PER_S // WB, wb, 0)


def _proj_body(x_ref, w_ref, b_ref, o_ref):
    o_ref[...] = (
        jnp.dot(x_ref[...], w_ref[...], preferred_element_type=jnp.float32)
        + b_ref[...])


def _proj(x, w, b):
    return pl.pallas_call(
        _proj_body,
        grid=(GRID,),
        in_specs=[
            pl.BlockSpec((RB, D), lambda i: (i, 0)),
            pl.BlockSpec((D, D), lambda i: (0, 0)),
            pl.BlockSpec((1, D), lambda i: (0, 0)),
        ],
        out_specs=pl.BlockSpec((RB, D), lambda i: (i, 0)),
        out_shape=jax.ShapeDtypeStruct((N, D), jnp.float32),
    )(x, w, b)


def _ln(v, g, b):
    mu = jnp.mean(v, axis=-1, keepdims=True)
    var = jnp.mean((v - mu) ** 2, axis=-1, keepdims=True)
    return (v - mu) * lax.rsqrt(var + 1e-5) * g + b


def _block_body(h_ref, p_ref, degp_ref, bt_ref, wl_ref, wr_ref, bb_ref,
                g_ref, beta_ref, gprev_ref, hout_ref, gout_ref, acc_ref):
    i = pl.program_id(0)
    hb = h_ref[...]
    psum = p_ref[0] + p_ref[1]
    ones_w = jnp.ones((DGB, 1), jnp.float32)
    deg_col = lax.dot_general(
        degp_ref[0], ones_w, (((0,), (0,)), ((), ())),
        preferred_element_type=jnp.float32)
    deg_col = jnp.maximum(deg_col, 1.0)
    agg = psum / deg_col
    node_conv = (
        jnp.dot(hb, wl_ref[...], preferred_element_type=jnp.float32)
        + jnp.dot(agg, wr_ref[...], preferred_element_type=jnp.float32)
        + bb_ref[...])
    onehot = (bt_ref[...] == lax.broadcasted_iota(jnp.int32, (RB, G), 1)
              ).astype(jnp.float32)
    gc = lax.dot_general(
        onehot, node_conv, (((0,), (0,)), ((), ())),
        preferred_element_type=jnp.float32)

    @pl.when(i == 0)
    def _():
        acc_ref[...] = gc

    @pl.when(i > 0)
    def _():
        acc_ref[...] += gc

    gamma = g_ref[...]
    beta = beta_ref[...]
    hout_ref[...] = jnp.maximum(_ln(node_conv + hb, gamma, beta), 0.0)

    @pl.when(i == GRID - 1)
    def _():
        gtot = acc_ref[...] + gprev_ref[...]
        gout_ref[...] = jnp.maximum(_ln(gtot, gamma, beta), 0.0)


def _block(h, p, degp, bt, wl, wr, bb, gamma, beta, gprev):
    return pl.pallas_call(
        _block_body,
        grid=(GRID,),
        in_specs=[
            pl.BlockSpec((RB, D), lambda i: (i, 0)),
            pl.BlockSpec((NC, RB, D), lambda i: (0, i, 0)),
            pl.BlockSpec((1, NW, RB), lambda i: (i, 0, 0)),
            pl.BlockSpec((RB, 1), lambda i: (i, 0)),
            pl.BlockSpec((D, D), lambda i: (0, 0)),
            pl.BlockSpec((D, D), lambda i: (0, 0)),
            pl.BlockSpec((1, D), lambda i: (0, 0)),
            pl.BlockSpec((1, D), lambda i: (0, 0)),
            pl.BlockSpec((1, D), lambda i: (0, 0)),
            pl.BlockSpec((G, D), lambda i: (0, 0)),
        ],
        out_specs=[
            pl.BlockSpec((RB, D), lambda i: (i, 0)),
            pl.BlockSpec((G, D), lambda i: (0, 0)),
        ],
        out_shape=[
            jax.ShapeDtypeStruct((N, D), jnp.float32),
            jax.ShapeDtypeStruct((G, D), jnp.float32),
        ],
        scratch_shapes=[pltpu.VMEM((G, D), jnp.float32)],
    )(h, p, degp, bt, wl, wr, bb, gamma, beta, gprev)


def kernel(x, edge_index, batch, W_fc, b_fc, Wl, Wr, bb, gamma, beta):
    src = edge_index[0].astype(jnp.int32)
    dst = edge_index[1].astype(jnp.int32)
    bt = batch.astype(jnp.int32).reshape(N, 1)
    zrows = jnp.zeros((WB, D), jnp.float32)
    zdeg = jnp.zeros((N,), jnp.float32)

    degp = _deg_sc(dst, zdeg).reshape(NW, GRID, RB).transpose(1, 0, 2)
    src3 = src.reshape(NW, CH, K)
    dst3 = dst.reshape(NW, CH, K)
    h = _proj(x, W_fc, b_fc.reshape(1, D))
    g = jnp.zeros((G, D), jnp.float32)
    for i in range(NUM_BLOCKS):
        p = _agg_sc(h, src3, dst3, zrows)
        h, g = _block(h, p, degp, bt, Wl[i], Wr[i], bb[i].reshape(1, D),
                      gamma[i].reshape(1, D), beta[i].reshape(1, D), g)
    return h, g
